# baseline (speedup 1.0000x reference)
import functools

import jax
import jax.numpy as jnp
from jax import lax
from jax.experimental import pallas as pl
from jax.experimental.pallas import tpu as pltpu
from jax.experimental.pallas import tpu_sc as plsc

_N = 10000
_NP = 10000
_E = 320000
_NC = 2
_NS = 16
_B = 80
_RC = _NP // _NS
_ZB = 25

_params = pltpu.CompilerParams(use_tc_tiling_on_sc=False)


@functools.cache
def _sc_mesh():
  return plsc.VectorSubcoreMesh(core_axis_name="c", subcore_axis_name="s",
                                num_cores=_NC, num_subcores=_NS)


def _sc_aggregate(feat, pk3, nblk, f, with_deg=False):
  out_type = [jax.ShapeDtypeStruct((_NC * _NP, f), jnp.float32)]
  scratch = [
      pltpu.VMEM((nblk, _B), jnp.int32),
      pltpu.VMEM((_B,), jnp.int32),
      pltpu.VMEM((_B,), jnp.int32),
      pltpu.VMEM((_B,), jnp.int32),
      pltpu.VMEM((_B,), jnp.int32),
      pltpu.VMEM((_B, f), jnp.float32),
      pltpu.VMEM((_B, f), jnp.float32),
      pltpu.VMEM_SHARED((_NP, f), jnp.float32),
      pltpu.SemaphoreType.DMA,
      pltpu.SemaphoreType.DMA,
      pltpu.SemaphoreType.DMA,
      pltpu.SemaphoreType.DMA,
  ]
  if with_deg:
    out_type.append(jax.ShapeDtypeStruct((_NC * _NP, 16), jnp.float32))
    scratch += [
        pltpu.VMEM((_B, 16), jnp.float32),
        pltpu.VMEM_SHARED((_NP, 16), jnp.float32),
    ]

  def body(*refs):
    if with_deg:
      (feat_h, pk_h, out_h, deg_h,
       pk_v, s0, d0, s1, d1, rows0, rows1, acc_s, sem0, sem1, sem2, sem3,
       ones_v, dacc_s) = refs
    else:
      (feat_h, pk_h, out_h,
       pk_v, s0, d0, s1, d1, rows0, rows1, acc_s, sem0, sem1,
       sem2, sem3) = refs
    c = lax.axis_index("c")
    s = lax.axis_index("s")
    wid = c * _NS + s
    pwid = wid if pk_h.shape[0] == _NC * _NS else s
    fref = feat_h.at[c] if len(feat_h.shape) == 3 else feat_h
    zero16 = jnp.zeros((16,), jnp.float32)

    @pl.loop(0, _B)
    def _(i):
      @pl.loop(0, f // 16)
      def _(k):
        rows0[i, pl.ds(k * 16, 16)] = zero16

    @pl.loop(0, _RC // _ZB)
    def _(i):
      pltpu.sync_copy(rows0.at[pl.ds(0, _ZB)],
                      acc_s.at[pl.ds(s * _RC + i * _ZB, _ZB)])

    if with_deg:
      @pl.loop(0, _B)
      def _(i):
        ones_v[i, pl.ds(0, 16)] = zero16

      @pl.loop(0, _RC // _ZB)
      def _(i):
        pltpu.sync_copy(ones_v.at[pl.ds(0, _ZB)],
                        dacc_s.at[pl.ds(s * _RC + i * _ZB, _ZB)])

    plsc.subcore_barrier()

    if with_deg:
      one16 = jnp.full((16,), 1.0, jnp.float32)

      @pl.loop(0, _B)
      def _(i):
        ones_v[i, pl.ds(0, 16)] = one16

    pltpu.sync_copy(pk_h.at[pwid], pk_v)
    mask15 = jnp.full((16,), 0x7FFF, jnp.int32)

    def unpack(j, sbuf, dbuf):
      @pl.loop(0, _B // 16)
      def _(k):
        v = pk_v[j, pl.ds(k * 16, 16)]
        sbuf[pl.ds(k * 16, 16)] = v & mask15
        dbuf[pl.ds(k * 16, 16)] = v >> 15

    def scat_start(rows_v, dbuf, sem):
      descs = [pltpu.async_copy(rows_v, acc_s.at[dbuf], sem, add=True)]
      if with_deg:
        descs.append(pltpu.async_copy(ones_v, dacc_s.at[dbuf], sem, add=True))
      return descs

    def wait_all(descs):
      for de in descs:
        de.wait()

    unpack(0, s0, d0)
    pltpu.async_copy(fref.at[s0], rows0, sem0)
    npair = (nblk - 1) // 2 * 2

    @pl.loop(0, npair, step=2)
    def _(j):
      unpack(j + 1, s1, d1)
      pltpu.async_copy(fref.at[s1], rows1, sem1)
      pltpu.make_async_copy(fref.at[s0], rows0, sem0).wait()
      sc0 = scat_start(rows0, d0, sem2)
      pltpu.make_async_copy(fref.at[s1], rows1, sem1).wait()
      sc1 = scat_start(rows1, d1, sem3)
      wait_all(sc0)
      unpack(j + 2, s0, d0)
      pltpu.async_copy(fref.at[s0], rows0, sem0)
      wait_all(sc1)

    if nblk % 2 == 0:
      unpack(nblk - 1, s1, d1)
      pltpu.async_copy(fref.at[s1], rows1, sem1)
      pltpu.make_async_copy(fref.at[s0], rows0, sem0).wait()
      sc0 = scat_start(rows0, d0, sem2)
      pltpu.make_async_copy(fref.at[s1], rows1, sem1).wait()
      sc1 = scat_start(rows1, d1, sem3)
      wait_all(sc0)
      wait_all(sc1)
    else:
      pltpu.make_async_copy(fref.at[s0], rows0, sem0).wait()
      wait_all(scat_start(rows0, d0, sem2))

    plsc.subcore_barrier()

    @pl.loop(0, _RC // _ZB)
    def _(i):
      r = s * _RC + i * _ZB
      pltpu.sync_copy(acc_s.at[pl.ds(r, _ZB)], rows0.at[pl.ds(0, _ZB)])
      pltpu.sync_copy(rows0.at[pl.ds(0, _ZB)],
                      out_h.at[pl.ds(c * _NP + r, _ZB)])
      if with_deg:
        pltpu.sync_copy(dacc_s.at[pl.ds(r, _ZB)], ones_v.at[pl.ds(0, _ZB)])
        pltpu.sync_copy(ones_v.at[pl.ds(0, _ZB)],
                        deg_h.at[pl.ds(c * _NP + r, _ZB)])

  k = pl.kernel(body, out_type=tuple(out_type), mesh=_sc_mesh(),
                scratch_types=scratch, compiler_params=_params)
  res = k(feat, pk3)
  return res if with_deg else res[0]


_R = 1000
_NB = _NP // _R


def _tc_layer1(agg, degs, x, WlT, WrT, b):
  def body(a0, a1, d0, d1, xb, wl, wr, bb, o):
    deg = jnp.maximum(d0[:, :1] + d1[:, :1], 1.0)
    mean = (a0[...] + a1[...]) / deg
    h = jnp.dot(mean, wl[...], preferred_element_type=jnp.float32)
    h = h + jnp.dot(xb[...], wr[...], preferred_element_type=jnp.float32)
    o[0] = jnp.maximum(h + bb[...], 0.0)

  return pl.pallas_call(
      body,
      grid=(2, _NB),
      in_specs=[
          pl.BlockSpec((_R, 128), lambda cc, i: (i, 0)),
          pl.BlockSpec((_R, 128), lambda cc, i: (i + _NB, 0)),
          pl.BlockSpec((_R, 16), lambda cc, i: (i, 0)),
          pl.BlockSpec((_R, 16), lambda cc, i: (i + _NB, 0)),
          pl.BlockSpec((_R, 128), lambda cc, i: (i, 0)),
          pl.BlockSpec((128, 128), lambda cc, i: (0, cc)),
          pl.BlockSpec((128, 128), lambda cc, i: (0, cc)),
          pl.BlockSpec((1, 128), lambda cc, i: (0, cc)),
      ],
      out_specs=pl.BlockSpec((1, _R, 128), lambda cc, i: (cc, i, 0)),
      out_shape=jax.ShapeDtypeStruct((_NC, _NP, 128), jnp.float32),
  )(agg, agg, degs, degs, x, WlT, WrT, b)


def _tc_layer23(agg2, degs, ht, W2lT, W2rT, b2, W3lT, W3rT, b3):
  def body(a0, a1, d0, d1, h0, h1, wl, wr, bb, wl3, wr3, b3b, t3o, r3o):
    deg = jnp.maximum(d0[:, :1] + d1[:, :1], 1.0)
    wl_ = wl[...]
    wr_ = wr[...]
    h2 = jnp.dot(a0[...] / deg, wl_[:128], preferred_element_type=jnp.float32)
    h2 = h2 + jnp.dot(a1[...] / deg, wl_[128:],
                      preferred_element_type=jnp.float32)
    h2 = h2 + jnp.dot(h0[0], wr_[:128], preferred_element_type=jnp.float32)
    h2 = h2 + jnp.dot(h1[0], wr_[128:], preferred_element_type=jnp.float32)
    h2 = jnp.maximum(h2 + bb[...], 0.0)
    t3o[...] = jnp.dot(h2, wl3[...], preferred_element_type=jnp.float32)
    r3o[...] = jnp.dot(h2, wr3[...],
                       preferred_element_type=jnp.float32) + b3b[...]

  return pl.pallas_call(
      body,
      grid=(_NB,),
      in_specs=[
          pl.BlockSpec((_R, 128), lambda i: (i, 0)),
          pl.BlockSpec((_R, 128), lambda i: (i + _NB, 0)),
          pl.BlockSpec((_R, 16), lambda i: (i, 0)),
          pl.BlockSpec((_R, 16), lambda i: (i + _NB, 0)),
          pl.BlockSpec((1, _R, 128), lambda i: (0, i, 0)),
          pl.BlockSpec((1, _R, 128), lambda i: (1, i, 0)),
          pl.BlockSpec((256, 256), lambda i: (0, 0)),
          pl.BlockSpec((256, 256), lambda i: (0, 0)),
          pl.BlockSpec((1, 256), lambda i: (0, 0)),
          pl.BlockSpec((256, 64), lambda i: (0, 0)),
          pl.BlockSpec((256, 64), lambda i: (0, 0)),
          pl.BlockSpec((1, 64), lambda i: (0, 0)),
      ],
      out_specs=[
          pl.BlockSpec((_R, 64), lambda i: (i, 0)),
          pl.BlockSpec((_R, 64), lambda i: (i, 0)),
      ],
      out_shape=[
          jax.ShapeDtypeStruct((_NP, 64), jnp.float32),
          jax.ShapeDtypeStruct((_NP, 64), jnp.float32),
      ],
  )(agg2, agg2, degs, degs, ht, ht, W2lT, W2rT, b2, W3lT, W3rT, b3)


def _tc_out(agg3, degs, r3):
  def body(a0, a1, d0, d1, rb, o):
    deg = jnp.maximum(d0[:, :1] + d1[:, :1], 1.0)
    z = (a0[...] + a1[...]) / deg + rb[...]
    m = jnp.max(z, axis=1, keepdims=True)
    e = jnp.exp(z - m)
    o[...] = (z - m) - jnp.log(jnp.sum(e, axis=1, keepdims=True))

  return pl.pallas_call(
      body,
      grid=(_NB,),
      in_specs=[
          pl.BlockSpec((_R, 64), lambda i: (i, 0)),
          pl.BlockSpec((_R, 64), lambda i: (i + _NB, 0)),
          pl.BlockSpec((_R, 16), lambda i: (i, 0)),
          pl.BlockSpec((_R, 16), lambda i: (i + _NB, 0)),
          pl.BlockSpec((_R, 64), lambda i: (i, 0)),
      ],
      out_specs=pl.BlockSpec((_R, 64), lambda i: (i, 0)),
      out_shape=jax.ShapeDtypeStruct((_NP, 64), jnp.float32),
  )(agg3, agg3, degs, degs, r3)


def kernel(x, edge_index, W1l, b1, W1r, W2l, b2, W2r, W3l, b3, W3r):
  pk = jnp.left_shift(edge_index[1], 15) | edge_index[0]

  nb1 = _E // (_NC * _NS) // _B
  pk3 = pk.reshape(_NC * _NS, nb1, _B)
  nb2 = _E // _NS // _B
  pk3_l2 = pk.reshape(_NS, nb2, _B)

  agg1, degs = _sc_aggregate(x, pk3, nblk=nb1, f=128, with_deg=True)
  ht = _tc_layer1(agg1, degs, x, W1l.T, W1r.T, b1.reshape(1, -1))
  agg2 = _sc_aggregate(ht, pk3_l2, nblk=nb2, f=128)
  t3, r3 = _tc_layer23(agg2, degs, ht, W2l.T, W2r.T, b2.reshape(1, -1),
                       W3l.T, W3r.T, b3.reshape(1, -1))
  agg3 = _sc_aggregate(t3, pk3, nblk=nb1, f=64)
  out = _tc_out(agg3, degs, r3)
  return out


# speedup vs baseline: 1.2477x; 1.2477x over previous
import functools

import jax
import jax.numpy as jnp
from jax import lax
from jax.experimental import pallas as pl
from jax.experimental.pallas import tpu as pltpu
from jax.experimental.pallas import tpu_sc as plsc

_N = 10000
_NP = 10000
_E = 320000
_NC = 2
_NS = 16
_B = 80
_RC = _NP // _NS
_ZB = 25

_params = pltpu.CompilerParams(use_tc_tiling_on_sc=False)


@functools.cache
def _sc_mesh():
  return plsc.VectorSubcoreMesh(core_axis_name="c", subcore_axis_name="s",
                                num_cores=_NC, num_subcores=_NS)


def _sc_aggregate(feat, pk3, nblk, f, with_deg=False):
  out_type = [jax.ShapeDtypeStruct((_NC * _NP, f), jnp.float32)]
  scratch = [
      pltpu.VMEM((nblk, _B), jnp.int32),
      pltpu.VMEM((_B,), jnp.int32),
      pltpu.VMEM((_B,), jnp.int32),
      pltpu.VMEM((_B,), jnp.int32),
      pltpu.VMEM((_B,), jnp.int32),
      pltpu.VMEM((_B, f), jnp.float32),
      pltpu.VMEM((_B, f), jnp.float32),
      pltpu.VMEM_SHARED((_NP, f), jnp.float32),
      pltpu.SemaphoreType.DMA,
      pltpu.SemaphoreType.DMA,
  ]
  if with_deg:
    out_type.append(jax.ShapeDtypeStruct((_NC * _NP, 16), jnp.float32))
    scratch += [
        pltpu.VMEM((_B, 16), jnp.float32),
        pltpu.VMEM_SHARED((_NP, 16), jnp.float32),
    ]

  def body(*refs):
    if with_deg:
      (feat_h, pk_h, out_h, deg_h,
       pk_v, s0, d0, s1, d1, rows0, rows1, acc_s, sem0, sem1,
       ones_v, dacc_s) = refs
    else:
      (feat_h, pk_h, out_h,
       pk_v, s0, d0, s1, d1, rows0, rows1, acc_s, sem0, sem1) = refs
    c = lax.axis_index("c")
    s = lax.axis_index("s")
    wid = c * _NS + s
    pwid = wid if pk_h.shape[0] == _NC * _NS else s
    fref = feat_h.at[c] if len(feat_h.shape) == 3 else feat_h
    zero16 = jnp.zeros((16,), jnp.float32)

    @pl.loop(0, _B)
    def _(i):
      @pl.loop(0, f // 16)
      def _(k):
        rows0[i, pl.ds(k * 16, 16)] = zero16

    @pl.loop(0, _RC // _ZB)
    def _(i):
      pltpu.sync_copy(rows0.at[pl.ds(0, _ZB)],
                      acc_s.at[pl.ds(s * _RC + i * _ZB, _ZB)])

    if with_deg:
      @pl.loop(0, _B)
      def _(i):
        ones_v[i, pl.ds(0, 16)] = zero16

      @pl.loop(0, _RC // _ZB)
      def _(i):
        pltpu.sync_copy(ones_v.at[pl.ds(0, _ZB)],
                        dacc_s.at[pl.ds(s * _RC + i * _ZB, _ZB)])

    plsc.subcore_barrier()

    if with_deg:
      one16 = jnp.full((16,), 1.0, jnp.float32)

      @pl.loop(0, _B)
      def _(i):
        ones_v[i, pl.ds(0, 16)] = one16

    pltpu.sync_copy(pk_h.at[pwid], pk_v)
    mask15 = jnp.full((16,), 0x7FFF, jnp.int32)

    def unpack(j, sbuf, dbuf):
      @pl.loop(0, _B // 16)
      def _(k):
        v = pk_v[j, pl.ds(k * 16, 16)]
        sbuf[pl.ds(k * 16, 16)] = v & mask15
        dbuf[pl.ds(k * 16, 16)] = v >> 15

    def scat(rows_v, dbuf):
      pltpu.sync_copy(rows_v, acc_s.at[dbuf], add=True)
      if with_deg:
        pltpu.sync_copy(ones_v, dacc_s.at[dbuf], add=True)

    unpack(0, s0, d0)
    pltpu.async_copy(fref.at[s0], rows0, sem0)
    npair = (nblk - 1) // 2 * 2

    @pl.loop(0, npair, step=2)
    def _(j):
      unpack(j + 1, s1, d1)
      pltpu.async_copy(fref.at[s1], rows1, sem1)
      pltpu.make_async_copy(fref.at[s0], rows0, sem0).wait()
      scat(rows0, d0)
      unpack(j + 2, s0, d0)
      pltpu.async_copy(fref.at[s0], rows0, sem0)
      pltpu.make_async_copy(fref.at[s1], rows1, sem1).wait()
      scat(rows1, d1)

    if nblk % 2 == 0:
      unpack(nblk - 1, s1, d1)
      pltpu.async_copy(fref.at[s1], rows1, sem1)
      pltpu.make_async_copy(fref.at[s0], rows0, sem0).wait()
      scat(rows0, d0)
      pltpu.make_async_copy(fref.at[s1], rows1, sem1).wait()
      scat(rows1, d1)
    else:
      pltpu.make_async_copy(fref.at[s0], rows0, sem0).wait()
      scat(rows0, d0)

    plsc.subcore_barrier()

    r = s * _RC
    pltpu.sync_copy(acc_s.at[pl.ds(r, _RC)], out_h.at[pl.ds(c * _NP + r, _RC)])
    if with_deg:
      pltpu.sync_copy(dacc_s.at[pl.ds(r, _RC)],
                      deg_h.at[pl.ds(c * _NP + r, _RC)])

  k = pl.kernel(body, out_type=tuple(out_type), mesh=_sc_mesh(),
                scratch_types=scratch, compiler_params=_params)
  res = k(feat, pk3)
  return res if with_deg else res[0]


_R = 1000
_NB = _NP // _R


def _tc_layer1(agg, degs, x, WlT, WrT, b):
  def body(a0, a1, d0, d1, xb, wl, wr, bb, o):
    deg = jnp.maximum(d0[:, :1] + d1[:, :1], 1.0)
    mean = (a0[...] + a1[...]) / deg
    h = jnp.dot(mean, wl[...], preferred_element_type=jnp.float32)
    h = h + jnp.dot(xb[...], wr[...], preferred_element_type=jnp.float32)
    o[0] = jnp.maximum(h + bb[...], 0.0)

  return pl.pallas_call(
      body,
      grid=(2, _NB),
      in_specs=[
          pl.BlockSpec((_R, 128), lambda cc, i: (i, 0)),
          pl.BlockSpec((_R, 128), lambda cc, i: (i + _NB, 0)),
          pl.BlockSpec((_R, 16), lambda cc, i: (i, 0)),
          pl.BlockSpec((_R, 16), lambda cc, i: (i + _NB, 0)),
          pl.BlockSpec((_R, 128), lambda cc, i: (i, 0)),
          pl.BlockSpec((128, 128), lambda cc, i: (0, cc)),
          pl.BlockSpec((128, 128), lambda cc, i: (0, cc)),
          pl.BlockSpec((1, 128), lambda cc, i: (0, cc)),
      ],
      out_specs=pl.BlockSpec((1, _R, 128), lambda cc, i: (cc, i, 0)),
      out_shape=jax.ShapeDtypeStruct((_NC, _NP, 128), jnp.float32),
  )(agg, agg, degs, degs, x, WlT, WrT, b)


def _tc_layer23(agg2, degs, ht, W2lT, W2rT, b2, W3lT, W3rT, b3):
  def body(a0, a1, d0, d1, h0, h1, wl, wr, bb, wl3, wr3, b3b, t3o, r3o):
    deg = jnp.maximum(d0[:, :1] + d1[:, :1], 1.0)
    wl_ = wl[...]
    wr_ = wr[...]
    h2 = jnp.dot(a0[...] / deg, wl_[:128], preferred_element_type=jnp.float32)
    h2 = h2 + jnp.dot(a1[...] / deg, wl_[128:],
                      preferred_element_type=jnp.float32)
    h2 = h2 + jnp.dot(h0[0], wr_[:128], preferred_element_type=jnp.float32)
    h2 = h2 + jnp.dot(h1[0], wr_[128:], preferred_element_type=jnp.float32)
    h2 = jnp.maximum(h2 + bb[...], 0.0)
    t3o[...] = jnp.dot(h2, wl3[...], preferred_element_type=jnp.float32)
    r3o[...] = jnp.dot(h2, wr3[...],
                       preferred_element_type=jnp.float32) + b3b[...]

  return pl.pallas_call(
      body,
      grid=(_NB,),
      in_specs=[
          pl.BlockSpec((_R, 128), lambda i: (i, 0)),
          pl.BlockSpec((_R, 128), lambda i: (i + _NB, 0)),
          pl.BlockSpec((_R, 16), lambda i: (i, 0)),
          pl.BlockSpec((_R, 16), lambda i: (i + _NB, 0)),
          pl.BlockSpec((1, _R, 128), lambda i: (0, i, 0)),
          pl.BlockSpec((1, _R, 128), lambda i: (1, i, 0)),
          pl.BlockSpec((256, 256), lambda i: (0, 0)),
          pl.BlockSpec((256, 256), lambda i: (0, 0)),
          pl.BlockSpec((1, 256), lambda i: (0, 0)),
          pl.BlockSpec((256, 64), lambda i: (0, 0)),
          pl.BlockSpec((256, 64), lambda i: (0, 0)),
          pl.BlockSpec((1, 64), lambda i: (0, 0)),
      ],
      out_specs=[
          pl.BlockSpec((_R, 64), lambda i: (i, 0)),
          pl.BlockSpec((_R, 64), lambda i: (i, 0)),
      ],
      out_shape=[
          jax.ShapeDtypeStruct((_NP, 64), jnp.float32),
          jax.ShapeDtypeStruct((_NP, 64), jnp.float32),
      ],
  )(agg2, agg2, degs, degs, ht, ht, W2lT, W2rT, b2, W3lT, W3rT, b3)


def _tc_out(agg3, degs, r3):
  def body(a0, a1, d0, d1, rb, o):
    deg = jnp.maximum(d0[:, :1] + d1[:, :1], 1.0)
    z = (a0[...] + a1[...]) / deg + rb[...]
    m = jnp.max(z, axis=1, keepdims=True)
    e = jnp.exp(z - m)
    o[...] = (z - m) - jnp.log(jnp.sum(e, axis=1, keepdims=True))

  return pl.pallas_call(
      body,
      grid=(_NB,),
      in_specs=[
          pl.BlockSpec((_R, 64), lambda i: (i, 0)),
          pl.BlockSpec((_R, 64), lambda i: (i + _NB, 0)),
          pl.BlockSpec((_R, 16), lambda i: (i, 0)),
          pl.BlockSpec((_R, 16), lambda i: (i + _NB, 0)),
          pl.BlockSpec((_R, 64), lambda i: (i, 0)),
      ],
      out_specs=pl.BlockSpec((_R, 64), lambda i: (i, 0)),
      out_shape=jax.ShapeDtypeStruct((_NP, 64), jnp.float32),
  )(agg3, agg3, degs, degs, r3)


def kernel(x, edge_index, W1l, b1, W1r, W2l, b2, W2r, W3l, b3, W3r):
  pk = jnp.left_shift(edge_index[1], 15) | edge_index[0]

  nb1 = _E // (_NC * _NS) // _B
  pk3 = pk.reshape(_NC * _NS, nb1, _B)
  nb2 = _E // _NS // _B
  pk3_l2 = pk.reshape(_NS, nb2, _B)

  agg1, degs = _sc_aggregate(x, pk3, nblk=nb1, f=128, with_deg=True)
  ht = _tc_layer1(agg1, degs, x, W1l.T, W1r.T, b1.reshape(1, -1))
  agg2 = _sc_aggregate(ht, pk3_l2, nblk=nb2, f=128)
  t3, r3 = _tc_layer23(agg2, degs, ht, W2l.T, W2r.T, b2.reshape(1, -1),
                       W3l.T, W3r.T, b3.reshape(1, -1))
  agg3 = _sc_aggregate(t3, pk3, nblk=nb1, f=64)
  out = _tc_out(agg3, degs, r3)
  return out
